# Initial kernel scaffold; baseline (speedup 1.0000x reference)
import jax
import jax.numpy as jnp
from jax.experimental import pallas as pl


def kernel(spike_input):
    raise NotImplementedError("write your pallas kernel here")


# speedup vs baseline: 65.1231x; 65.1231x over previous
import jax
import jax.numpy as jnp
from jax import lax
from jax.experimental import pallas as pl
from jax.experimental.pallas import tpu as pltpu
from jax.experimental.pallas import tpu_sc as plsc
import functools

K = 32
DT = 0.001
NC, NS, L = 2, 16, 16
NW = NC * NS
CHUNK = 128


def _body(in_hbm, out_hbm, buf, out_buf, cnt_buf):
    B, T, O = in_hbm.shape
    ngroups = O // L
    per_w = B // NW
    wid = lax.axis_index("s") * NC + lax.axis_index("c")

    lane_ids = [lax.iota(jnp.int32, L) + g * L for g in range(ngroups)]
    inf_vec = jnp.full((L,), jnp.inf, dtype=jnp.float32)

    for i in range(per_w):
        b = wid * per_w + i

        def init_row(k, _):
            for g in range(ngroups):
                out_buf[k, pl.ds(g * L, L)] = inf_vec
            return 0
        lax.fori_loop(0, K, init_row, 0)

        zero = jnp.zeros((L,), jnp.int32)
        for g in range(ngroups):
            cnt_buf[g, :] = zero

        def chunk_iter(c, _):
            mnv = cnt_buf[0, :]
            for g in range(1, ngroups):
                mnv = jnp.minimum(mnv, cnt_buf[g, :])
            mn = mnv[0]
            for l in range(1, L):
                mn = jnp.minimum(mn, mnv[l])

            @pl.when(mn < K)
            def _():
                t = pl.multiple_of(c * CHUNK, CHUNK)
                pltpu.sync_copy(in_hbm.at[b, pl.ds(t, CHUNK), :], buf)

                def step(j, cnts):
                    cnts = list(cnts)
                    tval = ((t + j).astype(jnp.float32) * DT)
                    val = lax.broadcast(tval, (L,))
                    for g in range(ngroups):
                        x = buf[j, pl.ds(g * L, L)]
                        m = x != 0.0
                        cg = cnts[g]
                        ok = jnp.logical_and(m, cg < K)
                        idx = jnp.minimum(cg, K - 1)
                        plsc.store_scatter(out_buf, [idx, lane_ids[g]],
                                           val, mask=ok)
                        cnts[g] = cg + m.astype(jnp.int32)
                    return tuple(cnts)

                cnts0 = tuple(cnt_buf[g, :] for g in range(ngroups))
                cnts = lax.fori_loop(0, CHUNK, step, cnts0)
                for g in range(ngroups):
                    cnt_buf[g, :] = cnts[g]

            return 0

        lax.fori_loop(0, T // CHUNK, chunk_iter, 0)
        pltpu.sync_copy(out_buf, out_hbm.at[b])


@jax.jit
def kernel(spike_input):
    B, T, O = spike_input.shape
    mesh = plsc.VectorSubcoreMesh(core_axis_name="c", subcore_axis_name="s",
                                  num_cores=NC, num_subcores=NS)
    f = pl.kernel(
        _body,
        out_type=jax.ShapeDtypeStruct((B, K, O), jnp.float32),
        mesh=mesh,
        scratch_types=[
            pltpu.VMEM((CHUNK, O), jnp.float32),
            pltpu.VMEM((K, O), jnp.float32),
            pltpu.VMEM((O // L, L), jnp.int32),
        ],
        compiler_params=pltpu.CompilerParams(needs_layout_passes=False),
    )
    return f(spike_input)


# speedup vs baseline: 66.7622x; 1.0252x over previous
import jax
import jax.numpy as jnp
from jax import lax
from jax.experimental import pallas as pl
from jax.experimental.pallas import tpu as pltpu
from jax.experimental.pallas import tpu_sc as plsc

K = 32
DT = 0.001
NC, NS, L = 2, 16, 16
NW = NC * NS
CHUNK = 128
UNROLL = 2


def _body(in_hbm, out_hbm, buf, out_buf, cnt_buf, done_smem,
          sem0, sem1, sem_out):
    B, T, O = in_hbm.shape
    ngroups = O // L
    per_w = B // NW
    wid = lax.axis_index("s") * NC + lax.axis_index("c")

    lane_ids = [lax.iota(jnp.int32, L) + g * L for g in range(ngroups)]
    inf_vec = jnp.full((L,), jnp.inf, dtype=jnp.float32)
    zero = jnp.zeros((L,), jnp.int32)
    in_sems = [sem0, sem1]

    for i in range(per_w):
        pltpu.async_copy(in_hbm.at[wid * per_w + i, pl.ds(0, CHUNK), :],
                         buf.at[i], in_sems[i])

    def process_chunk(slot, t):
        def step(jj, cnts):
            cnts = list(cnts)
            for u in range(UNROLL):
                j = jj * UNROLL + u
                tval = ((t + j).astype(jnp.float32) * DT)
                val = lax.broadcast(tval, (L,))
                for g in range(ngroups):
                    x = buf[slot, j, pl.ds(g * L, L)]
                    m = x != 0.0
                    cg = cnts[g]
                    ok = jnp.logical_and(m, cg < K)
                    idx = jnp.minimum(cg, K - 1)
                    plsc.store_scatter(out_buf.at[slot],
                                       [idx, lane_ids[g]], val, mask=ok)
                    cnts[g] = cg + m.astype(jnp.int32)
            return tuple(cnts)

        cnts0 = tuple(cnt_buf[g, :] for g in range(ngroups))
        cnts = lax.fori_loop(0, CHUNK // UNROLL, step, cnts0)
        for g in range(ngroups):
            cnt_buf[g, :] = cnts[g]
        mnv = cnts[0]
        for g in range(1, ngroups):
            mnv = jnp.minimum(mnv, cnts[g])
        mn = mnv[0]
        for l in range(1, L):
            mn = jnp.minimum(mn, mnv[l])
        done_smem[0] = (mn >= K).astype(jnp.int32)

    for i in range(per_w):
        b = wid * per_w + i

        def init_row(k, _):
            for g in range(ngroups):
                out_buf[i, k, pl.ds(g * L, L)] = inf_vec
            return 0
        lax.fori_loop(0, K, init_row, 0)
        for g in range(ngroups):
            cnt_buf[g, :] = zero

        pltpu.make_async_copy(in_hbm.at[b, pl.ds(0, CHUNK), :],
                              buf.at[i], in_sems[i]).wait()
        process_chunk(i, jnp.int32(0))

        def chunk_iter(c, _):
            @pl.when(done_smem[0] == 0)
            def _():
                t = pl.multiple_of(c * CHUNK, CHUNK)
                pltpu.sync_copy(in_hbm.at[b, pl.ds(t, CHUNK), :],
                                buf.at[i])
                process_chunk(i, t)
            return 0

        lax.fori_loop(1, T // CHUNK, chunk_iter, 0)
        pltpu.async_copy(out_buf.at[i], out_hbm.at[b], sem_out)

    for i in range(per_w):
        pltpu.make_async_copy(out_buf.at[i],
                              out_hbm.at[wid * per_w + i], sem_out).wait()


@jax.jit
def kernel(spike_input):
    B, T, O = spike_input.shape
    per_w = B // NW
    mesh = plsc.VectorSubcoreMesh(core_axis_name="c", subcore_axis_name="s",
                                  num_cores=NC, num_subcores=NS)
    f = pl.kernel(
        _body,
        out_type=jax.ShapeDtypeStruct((B, K, O), jnp.float32),
        mesh=mesh,
        scratch_types=[
            pltpu.VMEM((per_w, CHUNK, O), jnp.float32),
            pltpu.VMEM((per_w, K, O), jnp.float32),
            pltpu.VMEM((O // L, L), jnp.int32),
            pltpu.SMEM((1,), jnp.int32),
            pltpu.SemaphoreType.DMA,
            pltpu.SemaphoreType.DMA,
            pltpu.SemaphoreType.DMA,
        ],
        compiler_params=pltpu.CompilerParams(needs_layout_passes=False),
    )
    return f(spike_input)


# speedup vs baseline: 70.0841x; 1.0498x over previous
import jax
import jax.numpy as jnp
from jax import lax
from jax.experimental import pallas as pl
from jax.experimental.pallas import tpu as pltpu
from jax.experimental.pallas import tpu_sc as plsc

K = 32
DT = 0.001
NC, NS, L = 2, 16, 16
NW = NC * NS
CHUNK = 128
UNROLL = 4


def _body(in_hbm, out_hbm, buf, out_buf, cnt_buf, done_smem,
          sem0, sem1, sem_out):
    B, T, O = in_hbm.shape
    ngroups = O // L
    per_w = B // NW
    wid = lax.axis_index("s") * NC + lax.axis_index("c")

    lane_ids = [lax.iota(jnp.int32, L) + g * L for g in range(ngroups)]
    inf_vec = jnp.full((L,), jnp.inf, dtype=jnp.float32)
    zero = jnp.zeros((L,), jnp.int32)
    in_sems = [sem0, sem1]

    for i in range(per_w):
        pltpu.async_copy(in_hbm.at[wid * per_w + i, pl.ds(0, CHUNK), :],
                         buf.at[i], in_sems[i])

    def process_chunk(slot, t):
        cnts0 = tuple(cnt_buf[g, :] for g in range(ngroups))

        @plsc.parallel_loop(0, CHUNK, unroll=UNROLL, carry=cnts0)
        def cnts(j, cnts):
            cnts = list(cnts)
            tval = ((t + j).astype(jnp.float32) * DT)
            val = lax.broadcast(tval, (L,))
            for g in range(ngroups):
                x = buf[slot, j, pl.ds(g * L, L)]
                m = x != 0.0
                cg = cnts[g]
                ok = jnp.logical_and(m, cg < K)
                idx = jnp.minimum(cg, K - 1)
                plsc.store_scatter(out_buf.at[slot],
                                   [idx, lane_ids[g]], val, mask=ok)
                cnts[g] = cg + m.astype(jnp.int32)
            return tuple(cnts)
        for g in range(ngroups):
            cnt_buf[g, :] = cnts[g]
        mnv = cnts[0]
        for g in range(1, ngroups):
            mnv = jnp.minimum(mnv, cnts[g])
        mn = mnv[0]
        for l in range(1, L):
            mn = jnp.minimum(mn, mnv[l])
        done_smem[0] = (mn >= K).astype(jnp.int32)

    for i in range(per_w):
        b = wid * per_w + i

        def init_row(k, _):
            for g in range(ngroups):
                out_buf[i, k, pl.ds(g * L, L)] = inf_vec
            return 0
        lax.fori_loop(0, K, init_row, 0)
        for g in range(ngroups):
            cnt_buf[g, :] = zero

        pltpu.make_async_copy(in_hbm.at[b, pl.ds(0, CHUNK), :],
                              buf.at[i], in_sems[i]).wait()
        process_chunk(i, jnp.int32(0))

        def chunk_iter(c, _):
            @pl.when(done_smem[0] == 0)
            def _():
                t = pl.multiple_of(c * CHUNK, CHUNK)
                pltpu.sync_copy(in_hbm.at[b, pl.ds(t, CHUNK), :],
                                buf.at[i])
                process_chunk(i, t)
            return 0

        lax.fori_loop(1, T // CHUNK, chunk_iter, 0)
        pltpu.async_copy(out_buf.at[i], out_hbm.at[b], sem_out)

    for i in range(per_w):
        pltpu.make_async_copy(out_buf.at[i],
                              out_hbm.at[wid * per_w + i], sem_out).wait()


@jax.jit
def kernel(spike_input):
    B, T, O = spike_input.shape
    per_w = B // NW
    mesh = plsc.VectorSubcoreMesh(core_axis_name="c", subcore_axis_name="s",
                                  num_cores=NC, num_subcores=NS)
    f = pl.kernel(
        _body,
        out_type=jax.ShapeDtypeStruct((B, K, O), jnp.float32),
        mesh=mesh,
        scratch_types=[
            pltpu.VMEM((per_w, CHUNK, O), jnp.float32),
            pltpu.VMEM((per_w, K, O), jnp.float32),
            pltpu.VMEM((O // L, L), jnp.int32),
            pltpu.SMEM((1,), jnp.int32),
            pltpu.SemaphoreType.DMA,
            pltpu.SemaphoreType.DMA,
            pltpu.SemaphoreType.DMA,
        ],
        compiler_params=pltpu.CompilerParams(needs_layout_passes=False),
    )
    return f(spike_input)


# speedup vs baseline: 70.3482x; 1.0038x over previous
import jax
import jax.numpy as jnp
from jax import lax
from jax.experimental import pallas as pl
from jax.experimental.pallas import tpu as pltpu
from jax.experimental.pallas import tpu_sc as plsc

K = 32
DT = 0.001
NC, NS, L = 2, 16, 16
NW = NC * NS
CHUNK = 128
UNROLL = 8


def _body(in_hbm, out_hbm, buf, out_buf, cnt_buf, done_smem,
          sem0, sem1, sem_out):
    B, T, O = in_hbm.shape
    ngroups = O // L
    per_w = B // NW
    wid = lax.axis_index("s") * NC + lax.axis_index("c")

    lane_ids = [lax.iota(jnp.int32, L) + g * L for g in range(ngroups)]
    inf_vec = jnp.full((L,), jnp.inf, dtype=jnp.float32)
    zero = jnp.zeros((L,), jnp.int32)
    in_sems = [sem0, sem1]

    for i in range(per_w):
        pltpu.async_copy(in_hbm.at[wid * per_w + i, pl.ds(0, CHUNK), :],
                         buf.at[i], in_sems[i])

    def process_chunk(slot, t):
        cnts0 = tuple(cnt_buf[g, :] for g in range(ngroups))

        @plsc.parallel_loop(0, CHUNK, unroll=UNROLL, carry=cnts0)
        def cnts(j, cnts):
            cnts = list(cnts)
            tval = ((t + j).astype(jnp.float32) * DT)
            val = lax.broadcast(tval, (L,))
            for g in range(ngroups):
                x = buf[slot, j, pl.ds(g * L, L)]
                m = x != 0.0
                cg = cnts[g]
                ok = jnp.logical_and(m, cg < K)
                idx = jnp.minimum(cg, K - 1)
                plsc.store_scatter(out_buf.at[slot],
                                   [idx, lane_ids[g]], val, mask=ok)
                cnts[g] = cg + m.astype(jnp.int32)
            return tuple(cnts)
        for g in range(ngroups):
            cnt_buf[g, :] = cnts[g]
        mnv = cnts[0]
        for g in range(1, ngroups):
            mnv = jnp.minimum(mnv, cnts[g])
        mn = mnv[0]
        for l in range(1, L):
            mn = jnp.minimum(mn, mnv[l])
        done_smem[0] = (mn >= K).astype(jnp.int32)

    for i in range(per_w):
        b = wid * per_w + i

        for k in range(K):
            for g in range(ngroups):
                out_buf[i, k, pl.ds(g * L, L)] = inf_vec
        for g in range(ngroups):
            cnt_buf[g, :] = zero

        pltpu.make_async_copy(in_hbm.at[b, pl.ds(0, CHUNK), :],
                              buf.at[i], in_sems[i]).wait()
        process_chunk(i, jnp.int32(0))

        @pl.when(done_smem[0] == 0)
        def _():
            def chunk_iter(c, _):
                @pl.when(done_smem[0] == 0)
                def _():
                    t = pl.multiple_of(c * CHUNK, CHUNK)
                    pltpu.sync_copy(in_hbm.at[b, pl.ds(t, CHUNK), :],
                                    buf.at[i])
                    process_chunk(i, t)
                return 0

            lax.fori_loop(1, T // CHUNK, chunk_iter, 0)
        pltpu.async_copy(out_buf.at[i], out_hbm.at[b], sem_out)

    for i in range(per_w):
        pltpu.make_async_copy(out_buf.at[i],
                              out_hbm.at[wid * per_w + i], sem_out).wait()


@jax.jit
def kernel(spike_input):
    B, T, O = spike_input.shape
    per_w = B // NW
    mesh = plsc.VectorSubcoreMesh(core_axis_name="c", subcore_axis_name="s",
                                  num_cores=NC, num_subcores=NS)
    f = pl.kernel(
        _body,
        out_type=jax.ShapeDtypeStruct((B, K, O), jnp.float32),
        mesh=mesh,
        scratch_types=[
            pltpu.VMEM((per_w, CHUNK, O), jnp.float32),
            pltpu.VMEM((per_w, K, O), jnp.float32),
            pltpu.VMEM((O // L, L), jnp.int32),
            pltpu.SMEM((1,), jnp.int32),
            pltpu.SemaphoreType.DMA,
            pltpu.SemaphoreType.DMA,
            pltpu.SemaphoreType.DMA,
        ],
        compiler_params=pltpu.CompilerParams(needs_layout_passes=False),
    )
    return f(spike_input)
